# baseline (speedup 1.0000x reference)
import functools

import jax
import jax.numpy as jnp
from jax import lax
from jax.experimental import pallas as pl
from jax.experimental.pallas import tpu as pltpu
from jax.experimental.pallas import tpu_sc as plsc

EMBED = 64

_NC = 2
_NS = 16
_NW = _NC * _NS

_CHUNK = 128
_NBUF = 2


@functools.lru_cache(maxsize=None)
def _make_gather(B, V, D):
    assert B % (_NW * _CHUNK) == 0
    n_chunks = B // (_NW * _CHUNK)
    b_per_w = n_chunks * _CHUNK
    assert n_chunks > _NBUF

    mesh = plsc.VectorSubcoreMesh(core_axis_name="c", subcore_axis_name="s")

    @functools.partial(
        pl.kernel,
        out_type=jax.ShapeDtypeStruct((B, D), jnp.float32),
        mesh=mesh,
        scratch_types=[
            pltpu.VMEM((n_chunks, _CHUNK), jnp.int32),
            [pltpu.VMEM((_CHUNK, D), jnp.float32) for _ in range(_NBUF)],
            [pltpu.SemaphoreType.DMA for _ in range(_NBUF)],
        ],
        compiler_params=pltpu.CompilerParams(use_tc_tiling_on_sc=False),
    )
    def gather_kernel(idx_hbm, table_hbm, out_hbm, idx_v, bufs, sems):
        wid = lax.axis_index("s") * _NC + lax.axis_index("c")
        base_row = wid * b_per_w
        pltpu.sync_copy(idx_hbm.at[wid], idx_v)

        for b in range(_NBUF):
            pltpu.async_copy(table_hbm.at[idx_v.at[b]], bufs[b], sems[b])

        def step(i, carry):
            j0 = i * _NBUF
            for b in range(_NBUF):
                j = j0 + b
                pltpu.make_async_copy(
                    table_hbm.at[idx_v.at[j]], bufs[b], sems[b]).wait()
                pltpu.sync_copy(
                    bufs[b],
                    out_hbm.at[pl.ds(base_row + j * _CHUNK, _CHUNK)])
                pltpu.async_copy(
                    table_hbm.at[idx_v.at[j + _NBUF]], bufs[b], sems[b])
            return carry

        n_main = (n_chunks - _NBUF) // _NBUF
        lax.fori_loop(0, n_main, step, 0, unroll=False)

        for b in range(_NBUF):
            j = n_main * _NBUF + b
            pltpu.make_async_copy(
                table_hbm.at[idx_v.at[j]], bufs[b], sems[b]).wait()
            pltpu.sync_copy(
                bufs[b], out_hbm.at[pl.ds(base_row + j * _CHUNK, _CHUNK)])

    return gather_kernel


def kernel(x, token_table, codebook):
    batch, length = x.shape
    B = batch * length
    V, D = token_table.shape
    idx = x.reshape(_NW, B // (_NW * _CHUNK), _CHUNK).astype(jnp.int32)
    out = _make_gather(B, V, D)(idx, token_table)
    return out.reshape(batch, length, D)
